# Initial kernel scaffold; baseline (speedup 1.0000x reference)
import jax
import jax.numpy as jnp
from jax.experimental import pallas as pl


def kernel(cls_logits, cls_regress, proposals):
    raise NotImplementedError("write your pallas kernel here")


# speedup vs baseline: 12.6960x; 12.6960x over previous
import functools

import jax
import jax.numpy as jnp
from jax.experimental import pallas as pl
from jax.experimental.pallas import tpu as pltpu
import numpy as np

N = 20000
NP = 20480
RW = 160
LN = 128
K = 500
KP = 512
NCLS = 20
C = 21
OUT = 100
OP = 128
IOU_THR = 0.5
SCORE_THR = 0.05
MAX_RATIO = abs(float(np.log(0.016)))
CHUNK = 2048
NCHUNK = NP // CHUNK

_INT_MIN = np.int32(-2147483648)
_INT_MAX = np.int32(2147483647)

_dot = functools.partial(jnp.dot, preferred_element_type=jnp.float32,
                         precision=jax.lax.Precision.HIGHEST)


def _monotone_key(x):
    b = jax.lax.bitcast_convert_type(x, jnp.int32)
    return jnp.where(b < 0, b ^ jnp.int32(0x7FFFFFFF), b)


def _kth_threshold(key, k):
    def body(_, carry):
        lo, hi = carry
        mid = (lo >> 1) + (hi >> 1) + ((lo | hi) & 1)
        cnt = jnp.sum((key >= mid).astype(jnp.int32))
        ok = cnt >= k
        return jnp.where(ok, mid, lo), jnp.where(ok, hi, mid - 1)

    lo, _ = jax.lax.fori_loop(0, 32, body, (_INT_MIN, _INT_MAX))
    return lo


def _flat_excl_cumsum(m, su, sl):
    ex_row = _dot(m, su)
    rowtot = jnp.sum(m, axis=1, keepdims=True)
    rowoff = _dot(sl, rowtot)
    return rowoff + ex_row


def _lt_mats(r, l):
    su = (jax.lax.broadcasted_iota(jnp.int32, (l, l), 0)
          < jax.lax.broadcasted_iota(jnp.int32, (l, l), 1)).astype(jnp.float32)
    sl = (jax.lax.broadcasted_iota(jnp.int32, (r, r), 1)
          < jax.lax.broadcasted_iota(jnp.int32, (r, r), 0)).astype(jnp.float32)
    return su, sl


def _decode_body(p_ref, d_ref, b_ref):
    p = p_ref[...]
    d = d_ref[...]
    x1, y1, x2, y2 = p[0:1, :], p[1:2, :], p[2:3, :], p[3:4, :]
    w = x2 - x1
    h = y2 - y1
    cx = x1 + 0.5 * w
    cy = y1 + 0.5 * h
    dx = d[0:1, :] * 0.1
    dy = d[1:2, :] * 0.1
    dw = jnp.clip(d[2:3, :] * 0.2, -MAX_RATIO, MAX_RATIO)
    dh = jnp.clip(d[3:4, :] * 0.2, -MAX_RATIO, MAX_RATIO)
    pcx = cx + dx * w
    pcy = cy + dy * h
    pw = w * jnp.exp(dw)
    ph = h * jnp.exp(dh)
    b_ref[...] = jnp.concatenate(
        [pcx - 0.5 * pw, pcy - 0.5 * ph, pcx + 0.5 * pw, pcy + 0.5 * ph], axis=0)


def _class_body(s_ref, box_ref, os_ref, oi_ref, sup3_ref):
    s2d = s_ref[0]
    key = _monotone_key(s2d)

    t = _kth_threshold(key, K)
    c_gt = jnp.sum((key > t).astype(jnp.int32))
    need = (K - c_gt).astype(jnp.float32)

    su, sl = _lt_mats(RW, LN)
    tie = (key == t)
    tie_rank = _flat_excl_cumsum(tie.astype(jnp.float32), su, sl)
    sel = (key > t) | (tie & (tie_rank < need))
    self_ = sel.astype(jnp.float32)
    pos = _flat_excl_cumsum(self_, su, sl)

    rowi = jax.lax.broadcasted_iota(jnp.int32, (RW, LN), 0)
    lanei = jax.lax.broadcasted_iota(jnp.int32, (RW, LN), 1)
    idxf = (rowi * LN + lanei).astype(jnp.float32)

    pos_col = jnp.transpose(pos.reshape(1, NP))
    sel_col = jnp.transpose(self_.reshape(1, NP))
    kio = jax.lax.broadcasted_iota(jnp.int32, (1, KP), 1).astype(jnp.float32)

    acc = jnp.zeros((2, KP), jnp.float32)
    cnt = jnp.zeros((1, KP), jnp.float32)
    for tch in range(NCHUNK):
        r0 = tch * (CHUNK // LN)
        m = ((pos_col[tch * CHUNK:(tch + 1) * CHUNK, :] == kio)
             & (sel_col[tch * CHUNK:(tch + 1) * CHUNK, :] > 0.0)
             ).astype(jnp.float32)
        w2 = jnp.concatenate(
            [s2d[r0:r0 + CHUNK // LN, :].reshape(1, CHUNK),
             idxf[r0:r0 + CHUNK // LN, :].reshape(1, CHUNK)], axis=0)
        acc = acc + _dot(w2, m)
        cnt = cnt + jnp.sum(m, axis=0, keepdims=True)

    filled = cnt > 0.0
    score_c = jnp.where(filled, acc[0:1, :], -1e9)
    idx_c = jnp.where(filled, acc[1:2, :], 1e9)

    s_col = jnp.transpose(score_c)
    i_col = jnp.transpose(idx_c)
    gt = (s_col > score_c) | ((s_col == score_c) & (i_col < idx_c))
    rank = jnp.sum(gt.astype(jnp.float32), axis=0, keepdims=True)
    p = (jnp.transpose(rank) == kio).astype(jnp.float32)
    sorted_s = _dot(score_c, p)
    sorted_i = _dot(idx_c, p)

    tb = jnp.zeros((4, KP), jnp.float32)
    io_col = jax.lax.broadcasted_iota(jnp.int32, (CHUNK, 1), 0).astype(jnp.float32)
    for tch in range(NCHUNK):
        msrc = ((io_col + (tch * CHUNK)) == sorted_i).astype(jnp.float32)
        bch = box_ref[:, tch * CHUNK:(tch + 1) * CHUNK]
        tb = tb + _dot(bch, msrc)

    x1r, y1r, x2r, y2r = tb[0:1, :], tb[1:2, :], tb[2:3, :], tb[3:4, :]
    arear = jnp.maximum(x2r - x1r, 0.0) * jnp.maximum(y2r - y1r, 0.0)
    x1c = jnp.transpose(x1r)
    y1c = jnp.transpose(y1r)
    x2c = jnp.transpose(x2r)
    y2c = jnp.transpose(y2r)
    areac = jnp.transpose(arear)

    ix1 = jnp.maximum(x1c, x1r)
    iy1 = jnp.maximum(y1c, y1r)
    ix2 = jnp.minimum(x2c, x2r)
    iy2 = jnp.minimum(y2c, y2r)
    inter = jnp.maximum(ix2 - ix1, 0.0) * jnp.maximum(iy2 - iy1, 0.0)
    union = areac + arear - inter
    iou = inter / jnp.maximum(union, 1e-12)

    sli = jax.lax.broadcasted_iota(jnp.int32, (KP, KP), 0)
    lli = jax.lax.broadcasted_iota(jnp.int32, (KP, KP), 1)
    sup3_ref[0] = ((iou > IOU_THR) & (sli < lli)).astype(jnp.bfloat16)

    os_ref[0] = sorted_s
    oi_ref[0] = sorted_i


def _nms_body(ss_ref, sup_ref, kept_ref):
    s = ss_ref[...]
    lane = jax.lax.broadcasted_iota(jnp.int32, (NCLS, KP), 1)
    keep0 = (s > SCORE_THR).astype(jnp.float32)

    def body(ib, keep):
        r0 = pl.multiple_of(ib * 16, 16)
        blk = sup_ref[:, pl.ds(r0, 16), :]
        for j in range(16):
            i = ib * 16 + j
            row = blk[:, j, :].astype(jnp.float32)
            ki = jnp.sum(keep * (lane == i).astype(jnp.float32),
                         axis=1, keepdims=True)
            keep = keep * (1.0 - row * ki)
        return keep

    keep = jax.lax.fori_loop(0, KP // 16, body, keep0)
    kept_ref[...] = keep * s


def _merge_body(as_ref, ai_ref, lg_ref, box_ref, ol_ref, ob_ref):
    s = as_ref[...]
    key = _monotone_key(s)
    t = _kth_threshold(key, OUT)
    c_gt = jnp.sum((key > t).astype(jnp.int32))
    need = (OUT - c_gt).astype(jnp.float32)

    su, sl = _lt_mats(NCLS, KP)
    tie = (key == t)
    tie_rank = _flat_excl_cumsum(tie.astype(jnp.float32), su, sl)
    sel = (key > t) | (tie & (tie_rank < need))
    self_ = sel.astype(jnp.float32)
    pos = _flat_excl_cumsum(self_, su, sl)

    rowi = jax.lax.broadcasted_iota(jnp.int32, (NCLS, KP), 0)
    lanei = jax.lax.broadcasted_iota(jnp.int32, (NCLS, KP), 1)
    fpos = (rowi * KP + lanei).astype(jnp.float32)

    nflat = NCLS * KP
    pos_col = jnp.transpose(pos.reshape(1, nflat))
    sel_col = jnp.transpose(self_.reshape(1, nflat))
    oio = jax.lax.broadcasted_iota(jnp.int32, (1, OP), 1).astype(jnp.float32)

    acc = jnp.zeros((3, OP), jnp.float32)
    cnt = jnp.zeros((1, OP), jnp.float32)
    rows_per = CHUNK // KP
    for tch in range(nflat // CHUNK):
        r0 = tch * rows_per
        m = ((pos_col[tch * CHUNK:(tch + 1) * CHUNK, :] == oio)
             & (sel_col[tch * CHUNK:(tch + 1) * CHUNK, :] > 0.0)
             ).astype(jnp.float32)
        w3 = jnp.concatenate(
            [s[r0:r0 + rows_per, :].reshape(1, CHUNK),
             ai_ref[r0:r0 + rows_per, :].reshape(1, CHUNK),
             fpos[r0:r0 + rows_per, :].reshape(1, CHUNK)], axis=0)
        acc = acc + _dot(w3, m)
        cnt = cnt + jnp.sum(m, axis=0, keepdims=True)

    filled = cnt > 0.0
    score_c = jnp.where(filled, acc[0:1, :], -1e9)
    nidx_c = jnp.where(filled, acc[1:2, :], 1e9)
    fpos_c = jnp.where(filled, acc[2:3, :], 1e9)

    s_col = jnp.transpose(score_c)
    f_col = jnp.transpose(fpos_c)
    gt = (s_col > score_c) | ((s_col == score_c) & (f_col < fpos_c))
    rank = jnp.sum(gt.astype(jnp.float32), axis=0, keepdims=True)
    p = (jnp.transpose(rank) == oio).astype(jnp.float32)
    top_s = _dot(score_c, p)
    top_i = _dot(nidx_c, p)

    mask = (top_s > 0.0).astype(jnp.float32)

    accl = jnp.zeros((C, OP), jnp.float32)
    accb = jnp.zeros((4, OP), jnp.float32)
    io_col = jax.lax.broadcasted_iota(jnp.int32, (CHUNK, 1), 0).astype(jnp.float32)
    for tch in range(NCHUNK):
        mg = ((io_col + (tch * CHUNK)) == top_i).astype(jnp.float32)
        accl = accl + _dot(lg_ref[:, tch * CHUNK:(tch + 1) * CHUNK], mg)
        accb = accb + _dot(box_ref[:, tch * CHUNK:(tch + 1) * CHUNK], mg)

    ol_ref[...] = accl * mask
    ob_ref[...] = accb * mask


def kernel(cls_logits, cls_regress, proposals):
    logits_t = jnp.pad(cls_logits[0].T, ((0, 0), (0, NP - N)))
    scores3d = jnp.pad(cls_logits[0, :, 1:].T, ((0, 0), (0, NP - N)),
                       constant_values=-1.0).reshape(NCLS, RW, LN)
    props_t = jnp.pad(proposals[0].T, ((0, 0), (0, NP - N)))
    deltas_t = jnp.pad(cls_regress[0].T, ((0, 0), (0, NP - N)))

    boxes_t = pl.pallas_call(
        _decode_body,
        out_shape=jax.ShapeDtypeStruct((4, NP), jnp.float32),
    )(props_t, deltas_t)

    all_s, all_i, sup3 = pl.pallas_call(
        _class_body,
        grid=(NCLS,),
        in_specs=[
            pl.BlockSpec((1, RW, LN), lambda c: (c, 0, 0)),
            pl.BlockSpec((4, NP), lambda c: (0, 0)),
        ],
        out_specs=[
            pl.BlockSpec((1, 1, KP), lambda c: (c, 0, 0)),
            pl.BlockSpec((1, 1, KP), lambda c: (c, 0, 0)),
            pl.BlockSpec((1, KP, KP), lambda c: (c, 0, 0)),
        ],
        out_shape=[
            jax.ShapeDtypeStruct((NCLS, 1, KP), jnp.float32),
            jax.ShapeDtypeStruct((NCLS, 1, KP), jnp.float32),
            jax.ShapeDtypeStruct((NCLS, KP, KP), jnp.bfloat16),
        ],
    )(scores3d, boxes_t)

    kept = pl.pallas_call(
        _nms_body,
        out_shape=jax.ShapeDtypeStruct((NCLS, KP), jnp.float32),
    )(all_s.reshape(NCLS, KP), sup3)

    out_lt, out_bt = pl.pallas_call(
        _merge_body,
        out_shape=[
            jax.ShapeDtypeStruct((C, OP), jnp.float32),
            jax.ShapeDtypeStruct((4, OP), jnp.float32),
        ],
    )(kept, all_i.reshape(NCLS, KP), logits_t, boxes_t)

    out_logits = out_lt[:, :OUT].T.reshape(1, OUT, C)
    out_boxes = out_bt[:, :OUT].T.reshape(1, OUT, 4)
    return out_logits, out_boxes


# speedup vs baseline: 48.1133x; 3.7896x over previous
import functools

import jax
import jax.numpy as jnp
from jax.experimental import pallas as pl
from jax.experimental.pallas import tpu as pltpu
import numpy as np

N = 20000
NP = 20480
RW = 160
LN = 128
K = 500
KP = 512
NCLS = 20
C = 21
OUT = 100
OP = 128
IOU_THR = 0.5
SCORE_THR = 0.05
MAX_RATIO = abs(float(np.log(0.016)))
CHUNK = 2048
NCHUNK = NP // CHUNK

_INT_MIN = np.int32(-2147483648)
_INT_MAX = np.int32(2147483647)

_dotb = functools.partial(jnp.dot, preferred_element_type=jnp.float32)


def _split3(x):
    s1 = x.astype(jnp.bfloat16)
    r1 = x - s1.astype(jnp.float32)
    s2 = r1.astype(jnp.bfloat16)
    r2 = r1 - s2.astype(jnp.float32)
    s3 = r2.astype(jnp.bfloat16)
    return jnp.concatenate([s1, s2, s3], axis=0)


def _recombine3(g, r):
    return g[0:r] + g[r:2 * r] + g[2 * r:3 * r]


def _monotone_key(x):
    b = jax.lax.bitcast_convert_type(x, jnp.int32)
    return jnp.where(b < 0, b ^ jnp.int32(0x7FFFFFFF), b)


def _kth_threshold(key, k):
    def body(_, carry):
        lo, hi = carry
        mid = (lo >> 1) + (hi >> 1) + ((lo | hi) & 1)
        cnt = jnp.sum((key >= mid).astype(jnp.int32))
        ok = cnt >= k
        return jnp.where(ok, mid, lo), jnp.where(ok, hi, mid - 1)

    lo, _ = jax.lax.fori_loop(0, 32, body, (_INT_MIN, _INT_MAX))
    return lo


def _flat_excl_cumsum(m, su, sl):
    mb = m.astype(jnp.bfloat16)
    ex_row = _dotb(mb, su)
    rowtot = jnp.sum(m, axis=1, keepdims=True)
    rowoff = _dotb(sl, rowtot.astype(jnp.bfloat16))
    return rowoff + ex_row


def _lt_mats(r, l):
    su = (jax.lax.broadcasted_iota(jnp.int32, (l, l), 0)
          < jax.lax.broadcasted_iota(jnp.int32, (l, l), 1)).astype(jnp.bfloat16)
    sl = (jax.lax.broadcasted_iota(jnp.int32, (r, r), 1)
          < jax.lax.broadcasted_iota(jnp.int32, (r, r), 0)).astype(jnp.bfloat16)
    return su, sl


def _select_topk(key, k, su, sl):
    t = _kth_threshold(key, k)
    c_gt = jnp.sum((key > t).astype(jnp.int32))
    need = (k - c_gt).astype(jnp.float32)
    tie = (key == t)
    tie_rank = _flat_excl_cumsum(tie.astype(jnp.float32), su, sl)
    sel = (key > t) | (tie & (tie_rank < need))
    self_ = sel.astype(jnp.float32)
    pos = _flat_excl_cumsum(self_, su, sl)
    return jnp.where(sel, pos, -1.0)


def _decode_body(p_ref, d_ref, b_ref):
    p = p_ref[...]
    d = d_ref[...]
    x1, y1, x2, y2 = p[0:1, :], p[1:2, :], p[2:3, :], p[3:4, :]
    w = x2 - x1
    h = y2 - y1
    cx = x1 + 0.5 * w
    cy = y1 + 0.5 * h
    dx = d[0:1, :] * 0.1
    dy = d[1:2, :] * 0.1
    dw = jnp.clip(d[2:3, :] * 0.2, -MAX_RATIO, MAX_RATIO)
    dh = jnp.clip(d[3:4, :] * 0.2, -MAX_RATIO, MAX_RATIO)
    pcx = cx + dx * w
    pcy = cy + dy * h
    pw = w * jnp.exp(dw)
    ph = h * jnp.exp(dh)
    b_ref[...] = jnp.concatenate(
        [pcx - 0.5 * pw, pcy - 0.5 * ph, pcx + 0.5 * pw, pcy + 0.5 * ph], axis=0)


def _class_body(s_ref, box_ref, os_ref, oi_ref, sup3_ref):
    s2d = s_ref[0]
    key = _monotone_key(s2d)
    su, sl = _lt_mats(RW, LN)
    posq = _select_topk(key, K, su, sl)

    rowi = jax.lax.broadcasted_iota(jnp.int32, (RW, LN), 0)
    lanei = jax.lax.broadcasted_iota(jnp.int32, (RW, LN), 1)
    idxf = (rowi * LN + lanei).astype(jnp.float32)

    pos_col = jnp.transpose(posq.reshape(1, NP))
    kio = jax.lax.broadcasted_iota(jnp.int32, (1, KP), 1).astype(jnp.float32)

    acc = jnp.zeros((18, KP), jnp.float32)
    for tch in range(NCHUNK):
        r0 = tch * (CHUNK // LN)
        m = (pos_col[tch * CHUNK:(tch + 1) * CHUNK, :] == kio
             ).astype(jnp.bfloat16)
        q = jnp.concatenate(
            [s2d[r0:r0 + CHUNK // LN, :].reshape(1, CHUNK),
             idxf[r0:r0 + CHUNK // LN, :].reshape(1, CHUNK),
             box_ref[:, tch * CHUNK:(tch + 1) * CHUNK]], axis=0)
        acc = acc + _dotb(_split3(q), m)
    vals = _recombine3(acc, 6)

    fill = kio < float(K)
    score_c = jnp.where(fill, vals[0:1, :], -1e9)
    idx_c = jnp.where(fill, vals[1:2, :], 1e9)

    s_col = jnp.transpose(score_c)
    i_col = jnp.transpose(idx_c)
    gt = (s_col > score_c) | ((s_col == score_c) & (i_col < idx_c))
    rank = jnp.sum(gt.astype(jnp.float32), axis=0, keepdims=True)
    p = (jnp.transpose(rank) == kio).astype(jnp.bfloat16)
    allq = jnp.concatenate([score_c, idx_c, vals[2:6, :]], axis=0)
    sortedq = _recombine3(_dotb(_split3(allq), p), 6)
    sorted_s = sortedq[0:1, :]
    sorted_i = sortedq[1:2, :]

    x1r, y1r, x2r, y2r = (sortedq[2:3, :], sortedq[3:4, :],
                          sortedq[4:5, :], sortedq[5:6, :])
    arear = jnp.maximum(x2r - x1r, 0.0) * jnp.maximum(y2r - y1r, 0.0)
    x1c = jnp.transpose(x1r)
    y1c = jnp.transpose(y1r)
    x2c = jnp.transpose(x2r)
    y2c = jnp.transpose(y2r)
    areac = jnp.transpose(arear)

    ix1 = jnp.maximum(x1c, x1r)
    iy1 = jnp.maximum(y1c, y1r)
    ix2 = jnp.minimum(x2c, x2r)
    iy2 = jnp.minimum(y2c, y2r)
    inter = jnp.maximum(ix2 - ix1, 0.0) * jnp.maximum(iy2 - iy1, 0.0)
    union = areac + arear - inter
    iou = inter / jnp.maximum(union, 1e-12)

    sli = jax.lax.broadcasted_iota(jnp.int32, (KP, KP), 0)
    lli = jax.lax.broadcasted_iota(jnp.int32, (KP, KP), 1)
    sup3_ref[0] = ((iou > IOU_THR) & (sli < lli)).astype(jnp.bfloat16)

    os_ref[0] = sorted_s
    oi_ref[0] = sorted_i


def _nms_body(ss_ref, sup_ref, kept_ref):
    s = ss_ref[...]
    lane = jax.lax.broadcasted_iota(jnp.int32, (NCLS, KP), 1)
    keep0 = (s > SCORE_THR).astype(jnp.float32)

    def body(ib, keep):
        r0 = pl.multiple_of(ib * 16, 16)
        blk = sup_ref[:, pl.ds(r0, 16), :]
        for j in range(16):
            i = ib * 16 + j
            row = blk[:, j, :].astype(jnp.float32)
            ki = jnp.sum(keep * (lane == i).astype(jnp.float32),
                         axis=1, keepdims=True)
            keep = keep * (1.0 - row * ki)
        return keep

    keep = jax.lax.fori_loop(0, KP // 16, body, keep0)
    kept_ref[...] = keep * s


def _merge_body(as_ref, ai_ref, lg_ref, box_ref, ol_ref, ob_ref):
    s = as_ref[...]
    key = _monotone_key(s)
    su, sl = _lt_mats(NCLS, KP)
    posq = _select_topk(key, OUT, su, sl)

    rowi = jax.lax.broadcasted_iota(jnp.int32, (NCLS, KP), 0)
    lanei = jax.lax.broadcasted_iota(jnp.int32, (NCLS, KP), 1)
    fpos = (rowi * KP + lanei).astype(jnp.float32)

    nflat = NCLS * KP
    pos_col = jnp.transpose(posq.reshape(1, nflat))
    oio = jax.lax.broadcasted_iota(jnp.int32, (1, OP), 1).astype(jnp.float32)

    acc = jnp.zeros((9, OP), jnp.float32)
    rows_per = CHUNK // KP
    for tch in range(nflat // CHUNK):
        r0 = tch * rows_per
        m = (pos_col[tch * CHUNK:(tch + 1) * CHUNK, :] == oio
             ).astype(jnp.bfloat16)
        q = jnp.concatenate(
            [s[r0:r0 + rows_per, :].reshape(1, CHUNK),
             ai_ref[r0:r0 + rows_per, :].reshape(1, CHUNK),
             fpos[r0:r0 + rows_per, :].reshape(1, CHUNK)], axis=0)
        acc = acc + _dotb(_split3(q), m)
    vals = _recombine3(acc, 3)

    fill = oio < float(OUT)
    score_c = jnp.where(fill, vals[0:1, :], -1e9)
    nidx_c = jnp.where(fill, vals[1:2, :], 1e9)
    fpos_c = jnp.where(fill, vals[2:3, :], 1e9)

    s_col = jnp.transpose(score_c)
    f_col = jnp.transpose(fpos_c)
    gt = (s_col > score_c) | ((s_col == score_c) & (f_col < fpos_c))
    rank = jnp.sum(gt.astype(jnp.float32), axis=0, keepdims=True)
    p = (jnp.transpose(rank) == oio).astype(jnp.bfloat16)
    sorted2 = _recombine3(_dotb(_split3(jnp.concatenate(
        [score_c, nidx_c], axis=0)), p), 2)
    top_s = sorted2[0:1, :]
    top_i = sorted2[1:2, :]

    mask = (top_s > 0.0).astype(jnp.float32)

    acc2 = jnp.zeros((75, OP), jnp.float32)
    io_col = jax.lax.broadcasted_iota(jnp.int32, (CHUNK, 1), 0).astype(jnp.float32)
    for tch in range(NCHUNK):
        mg = ((io_col + (tch * CHUNK)) == top_i).astype(jnp.bfloat16)
        q = jnp.concatenate(
            [lg_ref[:, tch * CHUNK:(tch + 1) * CHUNK],
             box_ref[:, tch * CHUNK:(tch + 1) * CHUNK]], axis=0)
        acc2 = acc2 + _dotb(_split3(q), mg)
    gathered = _recombine3(acc2, 25)

    ol_ref[...] = gathered[0:C, :] * mask
    ob_ref[...] = gathered[C:C + 4, :] * mask


def kernel(cls_logits, cls_regress, proposals):
    logits_t = jnp.pad(cls_logits[0].T, ((0, 0), (0, NP - N)))
    scores3d = jnp.pad(cls_logits[0, :, 1:].T, ((0, 0), (0, NP - N)),
                       constant_values=-1.0).reshape(NCLS, RW, LN)
    props_t = jnp.pad(proposals[0].T, ((0, 0), (0, NP - N)))
    deltas_t = jnp.pad(cls_regress[0].T, ((0, 0), (0, NP - N)))

    boxes_t = pl.pallas_call(
        _decode_body,
        out_shape=jax.ShapeDtypeStruct((4, NP), jnp.float32),
    )(props_t, deltas_t)

    all_s, all_i, sup3 = pl.pallas_call(
        _class_body,
        grid=(NCLS,),
        in_specs=[
            pl.BlockSpec((1, RW, LN), lambda c: (c, 0, 0)),
            pl.BlockSpec((4, NP), lambda c: (0, 0)),
        ],
        out_specs=[
            pl.BlockSpec((1, 1, KP), lambda c: (c, 0, 0)),
            pl.BlockSpec((1, 1, KP), lambda c: (c, 0, 0)),
            pl.BlockSpec((1, KP, KP), lambda c: (c, 0, 0)),
        ],
        out_shape=[
            jax.ShapeDtypeStruct((NCLS, 1, KP), jnp.float32),
            jax.ShapeDtypeStruct((NCLS, 1, KP), jnp.float32),
            jax.ShapeDtypeStruct((NCLS, KP, KP), jnp.bfloat16),
        ],
    )(scores3d, boxes_t)

    kept = pl.pallas_call(
        _nms_body,
        out_shape=jax.ShapeDtypeStruct((NCLS, KP), jnp.float32),
    )(all_s.reshape(NCLS, KP), sup3)

    out_lt, out_bt = pl.pallas_call(
        _merge_body,
        out_shape=[
            jax.ShapeDtypeStruct((C, OP), jnp.float32),
            jax.ShapeDtypeStruct((4, OP), jnp.float32),
        ],
    )(kept, all_i.reshape(NCLS, KP), logits_t, boxes_t)

    out_logits = out_lt[:, :OUT].T.reshape(1, OUT, C)
    out_boxes = out_bt[:, :OUT].T.reshape(1, OUT, 4)
    return out_logits, out_boxes


# speedup vs baseline: 61.4664x; 1.2775x over previous
import functools

import jax
import jax.numpy as jnp
from jax.experimental import pallas as pl
from jax.experimental.pallas import tpu as pltpu
import numpy as np

N = 20000
NP = 20480
RW = 160
LN = 128
K = 500
KP = 512
NCLS = 20
C = 21
OUT = 100
OP = 128
IOU_THR = 0.5
SCORE_THR = 0.05
MAX_RATIO = abs(float(np.log(0.016)))
CHUNK = 2048
NCHUNK = NP // CHUNK

_INT_MIN = np.int32(-2147483648)
_INT_MAX = np.int32(2147483647)

_dotb = functools.partial(jnp.dot, preferred_element_type=jnp.float32)


def _split3(x):
    s1 = x.astype(jnp.bfloat16)
    r1 = x - s1.astype(jnp.float32)
    s2 = r1.astype(jnp.bfloat16)
    r2 = r1 - s2.astype(jnp.float32)
    s3 = r2.astype(jnp.bfloat16)
    return jnp.concatenate([s1, s2, s3], axis=0)


def _recombine3(g, r):
    return g[0:r] + g[r:2 * r] + g[2 * r:3 * r]


def _monotone_key(x):
    b = jax.lax.bitcast_convert_type(x, jnp.int32)
    return jnp.where(b < 0, b ^ jnp.int32(0x7FFFFFFF), b)


def _kth_threshold(key, k):
    def body(_, carry):
        lo, hi = carry
        mid = (lo >> 1) + (hi >> 1) + ((lo | hi) & 1)
        cnt = jnp.sum((key >= mid).astype(jnp.int32))
        ok = cnt >= k
        return jnp.where(ok, mid, lo), jnp.where(ok, hi, mid - 1)

    lo, _ = jax.lax.fori_loop(0, 32, body, (_INT_MIN, _INT_MAX))
    return lo


def _flat_excl_cumsum(m, su, sl):
    mb = m.astype(jnp.bfloat16)
    ex_row = _dotb(mb, su)
    rowtot = jnp.sum(m, axis=1, keepdims=True)
    rowoff = _dotb(sl, rowtot.astype(jnp.bfloat16))
    return rowoff + ex_row


def _lt_mats(r, l):
    su = (jax.lax.broadcasted_iota(jnp.int32, (l, l), 0)
          < jax.lax.broadcasted_iota(jnp.int32, (l, l), 1)).astype(jnp.bfloat16)
    sl = (jax.lax.broadcasted_iota(jnp.int32, (r, r), 1)
          < jax.lax.broadcasted_iota(jnp.int32, (r, r), 0)).astype(jnp.bfloat16)
    return su, sl


def _select_topk(key, k, su, sl, t=None, need=None):
    if t is None:
        t = _kth_threshold(key, k)
        c_gt = jnp.sum((key > t).astype(jnp.int32))
        need = (k - c_gt).astype(jnp.float32)
    tie = (key == t)
    tie_rank = _flat_excl_cumsum(tie.astype(jnp.float32), su, sl)
    sel = (key > t) | (tie & (tie_rank < need))
    self_ = sel.astype(jnp.float32)
    pos = _flat_excl_cumsum(self_, su, sl)
    return jnp.where(sel, pos, -1.0)


def _prep_body(p_ref, d_ref, s3_ref, b_ref, thr_ref, need_ref):
    p = p_ref[...]
    d = d_ref[...]
    x1, y1, x2, y2 = p[0:1, :], p[1:2, :], p[2:3, :], p[3:4, :]
    w = x2 - x1
    h = y2 - y1
    cx = x1 + 0.5 * w
    cy = y1 + 0.5 * h
    dx = d[0:1, :] * 0.1
    dy = d[1:2, :] * 0.1
    dw = jnp.clip(d[2:3, :] * 0.2, -MAX_RATIO, MAX_RATIO)
    dh = jnp.clip(d[3:4, :] * 0.2, -MAX_RATIO, MAX_RATIO)
    pcx = cx + dx * w
    pcy = cy + dy * h
    pw = w * jnp.exp(dw)
    ph = h * jnp.exp(dh)
    b_ref[...] = jnp.concatenate(
        [pcx - 0.5 * pw, pcy - 0.5 * ph, pcx + 0.5 * pw, pcy + 0.5 * ph], axis=0)

    key = _monotone_key(s3_ref[...])

    def body(_, carry):
        lo, hi = carry
        mid = (lo >> 1) + (hi >> 1) + ((lo | hi) & 1)
        cnt = jnp.sum((key >= mid).astype(jnp.int32), axis=(1, 2), keepdims=True)
        ok = cnt >= K
        return jnp.where(ok, mid, lo), jnp.where(ok, hi, mid - 1)

    init = (jnp.full((NCLS, 1, LN), _INT_MIN, jnp.int32),
            jnp.full((NCLS, 1, LN), _INT_MAX, jnp.int32))
    t, _ = jax.lax.fori_loop(0, 32, body, init)
    c_gt = jnp.sum((key > t).astype(jnp.int32), axis=(1, 2), keepdims=True)
    thr_ref[...] = t
    need_ref[...] = jnp.broadcast_to((K - c_gt).astype(jnp.float32),
                                     (NCLS, 1, LN))


def _class_body(s_ref, box_ref, thr_ref, need_ref, os_ref, oi_ref, sup3_ref):
    s2d = s_ref[0]
    key = _monotone_key(s2d)
    su, sl = _lt_mats(RW, LN)
    posq = _select_topk(key, K, su, sl,
                        t=thr_ref[0], need=need_ref[0])

    rowi = jax.lax.broadcasted_iota(jnp.int32, (RW, LN), 0)
    lanei = jax.lax.broadcasted_iota(jnp.int32, (RW, LN), 1)
    idxf = (rowi * LN + lanei).astype(jnp.float32)

    pos_col = jnp.transpose(posq.reshape(1, NP))
    kio = jax.lax.broadcasted_iota(jnp.int32, (1, KP), 1).astype(jnp.float32)

    acc = jnp.zeros((18, KP), jnp.float32)
    for tch in range(NCHUNK):
        r0 = tch * (CHUNK // LN)
        m = (pos_col[tch * CHUNK:(tch + 1) * CHUNK, :] == kio
             ).astype(jnp.bfloat16)
        q = jnp.concatenate(
            [s2d[r0:r0 + CHUNK // LN, :].reshape(1, CHUNK),
             idxf[r0:r0 + CHUNK // LN, :].reshape(1, CHUNK),
             box_ref[:, tch * CHUNK:(tch + 1) * CHUNK]], axis=0)
        acc = acc + _dotb(_split3(q), m)
    vals = _recombine3(acc, 6)

    fill = kio < float(K)
    score_c = jnp.where(fill, vals[0:1, :], -1e9)
    idx_c = jnp.where(fill, vals[1:2, :], 1e9)

    s_col = jnp.transpose(score_c)
    i_col = jnp.transpose(idx_c)
    gt = (s_col > score_c) | ((s_col == score_c) & (i_col < idx_c))
    rank = jnp.sum(gt.astype(jnp.float32), axis=0, keepdims=True)
    p = (jnp.transpose(rank) == kio).astype(jnp.bfloat16)
    allq = jnp.concatenate([score_c, idx_c, vals[2:6, :]], axis=0)
    sortedq = _recombine3(_dotb(_split3(allq), p), 6)
    sorted_s = sortedq[0:1, :]
    sorted_i = sortedq[1:2, :]

    x1r, y1r, x2r, y2r = (sortedq[2:3, :], sortedq[3:4, :],
                          sortedq[4:5, :], sortedq[5:6, :])
    arear = jnp.maximum(x2r - x1r, 0.0) * jnp.maximum(y2r - y1r, 0.0)
    x1c = jnp.transpose(x1r)
    y1c = jnp.transpose(y1r)
    x2c = jnp.transpose(x2r)
    y2c = jnp.transpose(y2r)
    areac = jnp.transpose(arear)

    ix1 = jnp.maximum(x1c, x1r)
    iy1 = jnp.maximum(y1c, y1r)
    ix2 = jnp.minimum(x2c, x2r)
    iy2 = jnp.minimum(y2c, y2r)
    inter = jnp.maximum(ix2 - ix1, 0.0) * jnp.maximum(iy2 - iy1, 0.0)
    union = areac + arear - inter
    iou = inter / jnp.maximum(union, 1e-12)

    sli = jax.lax.broadcasted_iota(jnp.int32, (KP, KP), 0)
    lli = jax.lax.broadcasted_iota(jnp.int32, (KP, KP), 1)
    sup3_ref[0] = ((iou > IOU_THR) & (sli < lli)).astype(jnp.bfloat16)

    os_ref[0] = sorted_s
    oi_ref[0] = sorted_i


def _nms_merge_body(ss_ref, sup_ref, ai_ref, lg_ref, box_ref, ol_ref, ob_ref):
    sorted_s = ss_ref[...]
    lane = jax.lax.broadcasted_iota(jnp.int32, (NCLS, KP), 1)
    keep0 = (sorted_s > SCORE_THR).astype(jnp.float32)

    def nbody(ib, keep):
        r0 = pl.multiple_of(ib * 16, 16)
        blk = sup_ref[:, pl.ds(r0, 16), :]
        for j in range(16):
            i = ib * 16 + j
            row = blk[:, j, :].astype(jnp.float32)
            ki = jnp.sum(keep * (lane == i).astype(jnp.float32),
                         axis=1, keepdims=True)
            keep = keep * (1.0 - row * ki)
        return keep

    keep = jax.lax.fori_loop(0, KP // 16, nbody, keep0)
    s = keep * sorted_s
    key = _monotone_key(s)
    su, sl = _lt_mats(NCLS, KP)
    posq = _select_topk(key, OUT, su, sl)

    rowi = jax.lax.broadcasted_iota(jnp.int32, (NCLS, KP), 0)
    lanei = jax.lax.broadcasted_iota(jnp.int32, (NCLS, KP), 1)
    fpos = (rowi * KP + lanei).astype(jnp.float32)

    nflat = NCLS * KP
    pos_col = jnp.transpose(posq.reshape(1, nflat))
    oio = jax.lax.broadcasted_iota(jnp.int32, (1, OP), 1).astype(jnp.float32)

    acc = jnp.zeros((9, OP), jnp.float32)
    rows_per = CHUNK // KP
    for tch in range(nflat // CHUNK):
        r0 = tch * rows_per
        m = (pos_col[tch * CHUNK:(tch + 1) * CHUNK, :] == oio
             ).astype(jnp.bfloat16)
        q = jnp.concatenate(
            [s[r0:r0 + rows_per, :].reshape(1, CHUNK),
             ai_ref[r0:r0 + rows_per, :].reshape(1, CHUNK),
             fpos[r0:r0 + rows_per, :].reshape(1, CHUNK)], axis=0)
        acc = acc + _dotb(_split3(q), m)
    vals = _recombine3(acc, 3)

    fill = oio < float(OUT)
    score_c = jnp.where(fill, vals[0:1, :], -1e9)
    nidx_c = jnp.where(fill, vals[1:2, :], 1e9)
    fpos_c = jnp.where(fill, vals[2:3, :], 1e9)

    s_col = jnp.transpose(score_c)
    f_col = jnp.transpose(fpos_c)
    gt = (s_col > score_c) | ((s_col == score_c) & (f_col < fpos_c))
    rank = jnp.sum(gt.astype(jnp.float32), axis=0, keepdims=True)
    p = (jnp.transpose(rank) == oio).astype(jnp.bfloat16)
    sorted2 = _recombine3(_dotb(_split3(jnp.concatenate(
        [score_c, nidx_c], axis=0)), p), 2)
    top_s = sorted2[0:1, :]
    top_i = sorted2[1:2, :]

    mask = (top_s > 0.0).astype(jnp.float32)

    acc2 = jnp.zeros((75, OP), jnp.float32)
    io_col = jax.lax.broadcasted_iota(jnp.int32, (CHUNK, 1), 0).astype(jnp.float32)
    for tch in range(NCHUNK):
        mg = ((io_col + (tch * CHUNK)) == top_i).astype(jnp.bfloat16)
        q = jnp.concatenate(
            [lg_ref[:, tch * CHUNK:(tch + 1) * CHUNK],
             box_ref[:, tch * CHUNK:(tch + 1) * CHUNK]], axis=0)
        acc2 = acc2 + _dotb(_split3(q), mg)
    gathered = _recombine3(acc2, 25)

    ol_ref[...] = gathered[0:C, :] * mask
    ob_ref[...] = gathered[C:C + 4, :] * mask


def kernel(cls_logits, cls_regress, proposals):
    logits_t = jnp.pad(cls_logits[0].T, ((0, 0), (0, NP - N)))
    scores3d = jnp.pad(cls_logits[0, :, 1:].T, ((0, 0), (0, NP - N)),
                       constant_values=-1.0).reshape(NCLS, RW, LN)
    props_t = jnp.pad(proposals[0].T, ((0, 0), (0, NP - N)))
    deltas_t = jnp.pad(cls_regress[0].T, ((0, 0), (0, NP - N)))

    boxes_t, thr3, need3 = pl.pallas_call(
        _prep_body,
        out_shape=[
            jax.ShapeDtypeStruct((4, NP), jnp.float32),
            jax.ShapeDtypeStruct((NCLS, 1, LN), jnp.int32),
            jax.ShapeDtypeStruct((NCLS, 1, LN), jnp.float32),
        ],
    )(props_t, deltas_t, scores3d)

    all_s, all_i, sup3 = pl.pallas_call(
        _class_body,
        grid=(NCLS,),
        in_specs=[
            pl.BlockSpec((1, RW, LN), lambda c: (c, 0, 0)),
            pl.BlockSpec((4, NP), lambda c: (0, 0)),
            pl.BlockSpec((1, 1, LN), lambda c: (c, 0, 0)),
            pl.BlockSpec((1, 1, LN), lambda c: (c, 0, 0)),
        ],
        out_specs=[
            pl.BlockSpec((1, 1, KP), lambda c: (c, 0, 0)),
            pl.BlockSpec((1, 1, KP), lambda c: (c, 0, 0)),
            pl.BlockSpec((1, KP, KP), lambda c: (c, 0, 0)),
        ],
        out_shape=[
            jax.ShapeDtypeStruct((NCLS, 1, KP), jnp.float32),
            jax.ShapeDtypeStruct((NCLS, 1, KP), jnp.float32),
            jax.ShapeDtypeStruct((NCLS, KP, KP), jnp.bfloat16),
        ],
    )(scores3d, boxes_t, thr3, need3)

    out_lt, out_bt = pl.pallas_call(
        _nms_merge_body,
        out_shape=[
            jax.ShapeDtypeStruct((C, OP), jnp.float32),
            jax.ShapeDtypeStruct((4, OP), jnp.float32),
        ],
    )(all_s.reshape(NCLS, KP), sup3, all_i.reshape(NCLS, KP),
      logits_t, boxes_t)

    out_logits = out_lt[:, :OUT].T.reshape(1, OUT, C)
    out_boxes = out_bt[:, :OUT].T.reshape(1, OUT, 4)
    return out_logits, out_boxes
